# baseline (speedup 1.0000x reference)
import jax
import jax.numpy as jnp
from jax import lax
from jax.experimental import pallas as pl
from jax.experimental.pallas import tpu as pltpu
from jax.experimental.pallas import tpu_sc as plsc

DIM = 50
DIM_PAD = 56
NC, NS = 2, 16
NW = NC * NS
CHUNK = 1024
SPLIT = 4


def _gather_body(table_hbm, idx_hbm, out_hbm, idx_v, rows_v, sem):
    wid = lax.axis_index("s") * NC + lax.axis_index("c")
    n = idx_hbm.shape[0]
    b_per_w = n // NW
    n_chunks = b_per_w // CHUNK
    base = wid * b_per_w

    def body(g, carry):
        off = base + g * CHUNK
        pltpu.sync_copy(idx_hbm.at[pl.ds(off, CHUNK)], idx_v)
        pltpu.async_copy(table_hbm.at[idx_v], rows_v, sem).wait()
        pltpu.sync_copy(rows_v, out_hbm.at[pl.ds(off, CHUNK)])
        return carry

    lax.fori_loop(0, n_chunks, body, 0)


def kernel(x, table):
    B, L = x.shape
    n = B * L
    np_ = n // SPLIT
    table_p = jnp.pad(table, ((0, 0), (0, DIM_PAD - DIM)))
    mesh = plsc.VectorSubcoreMesh(
        core_axis_name="c", subcore_axis_name="s",
        num_cores=NC, num_subcores=NS)
    call = pl.kernel(
        _gather_body,
        out_type=jax.ShapeDtypeStruct((np_, DIM_PAD), jnp.float32),
        mesh=mesh,
        scratch_types=[
            pltpu.VMEM((CHUNK,), jnp.int32),
            pltpu.VMEM((CHUNK, DIM_PAD), jnp.float32),
            pltpu.SemaphoreType.DMA,
        ],
        compiler_params=pltpu.CompilerParams(use_tc_tiling_on_sc=False),
    )
    idx = x.reshape(SPLIT, np_)
    parts = [call(table_p, idx[k])[:, :DIM] for k in range(SPLIT)]
    return jnp.concatenate(parts, axis=0).reshape(B, L, DIM)


# speedup vs baseline: 5.0308x; 5.0308x over previous
import jax
import jax.numpy as jnp
from jax import lax
from jax.experimental import pallas as pl
from jax.experimental.pallas import tpu as pltpu
from jax.experimental.pallas import tpu_sc as plsc

DIM = 50
DIM_PAD = 56
NC, NS = 2, 16
NW = NC * NS
CHUNK = 1024


def _gather_body(table_hbm, idx_hbm, out_hbm,
                 idx_v0, idx_v1, rows_v0, rows_v1,
                 sem_g0, sem_g1, sem_o0, sem_o1):
    wid = lax.axis_index("s") * NC + lax.axis_index("c")
    n = idx_hbm.shape[0]
    b_per_w = n // NW
    n_chunks = b_per_w // CHUNK
    base = wid * b_per_w
    idx_bufs = (idx_v0, idx_v1)
    row_bufs = (rows_v0, rows_v1)
    sem_g = (sem_g0, sem_g1)
    sem_o = (sem_o0, sem_o1)

    def start_gather(g, b):
        off = base + g * CHUNK
        pltpu.sync_copy(idx_hbm.at[pl.ds(off, CHUNK)], idx_bufs[b])
        pltpu.make_async_copy(
            table_hbm.at[idx_bufs[b]], row_bufs[b], sem_g[b]).start()

    def out_copy(g, b):
        off = base + g * CHUNK
        return pltpu.make_async_copy(
            row_bufs[b], out_hbm.at[pl.ds(off, CHUNK)], sem_o[b])

    start_gather(0, 0)

    def body(g2, carry):
        for b in (0, 1):
            g = g2 * 2 + b
            pltpu.make_async_copy(
                table_hbm.at[idx_bufs[b]], row_bufs[b], sem_g[b]).wait()

            @pl.when(g + 1 < n_chunks)
            def _():
                @pl.when(g >= 1)
                def _():
                    out_copy(g - 1, 1 - b).wait()
                start_gather(g + 1, 1 - b)

            out_copy(g, b).start()
        return carry

    lax.fori_loop(0, n_chunks // 2, body, 0)
    out_copy(n_chunks - 2, n_chunks % 2).wait()
    out_copy(n_chunks - 1, 1 - n_chunks % 2).wait()


def kernel(x, table):
    B, L = x.shape
    n = B * L
    idx = x.reshape(n)
    table_p = jnp.pad(table, ((0, 0), (0, DIM_PAD - DIM)))
    mesh = plsc.VectorSubcoreMesh(
        core_axis_name="c", subcore_axis_name="s",
        num_cores=NC, num_subcores=NS)
    out = pl.kernel(
        _gather_body,
        out_type=jax.ShapeDtypeStruct((n, DIM_PAD), jnp.float32),
        mesh=mesh,
        scratch_types=[
            pltpu.VMEM((CHUNK,), jnp.int32),
            pltpu.VMEM((CHUNK,), jnp.int32),
            pltpu.VMEM((CHUNK, DIM_PAD), jnp.float32),
            pltpu.VMEM((CHUNK, DIM_PAD), jnp.float32),
            pltpu.SemaphoreType.DMA,
            pltpu.SemaphoreType.DMA,
            pltpu.SemaphoreType.DMA,
            pltpu.SemaphoreType.DMA,
        ],
        compiler_params=pltpu.CompilerParams(use_tc_tiling_on_sc=False),
    )(table_p, idx)
    return out[:, :DIM].reshape(B, L, DIM)
